# baseline (speedup 1.0000x reference)
import jax
import jax.numpy as jnp
from jax.experimental import pallas as pl
from jax.experimental.pallas import tpu as pltpu

_C, _T, _H, _W = 3, 32, 224, 224
_S = _T // 4
_IDX = tuple((j * (_T - 1)) // (_S - 1) for j in range(_S))


def _body(in_ref, fast_ref, slow_ref, sem_fast, sem_slow):
    fast_dma = pltpu.make_async_copy(in_ref, fast_ref, sem_fast)
    fast_dma.start()
    slow_dmas = []
    for c in range(_C):
        for j, t in enumerate(_IDX):
            d = pltpu.make_async_copy(in_ref.at[c, t], slow_ref.at[c, j], sem_slow)
            d.start()
            slow_dmas.append(d)
    for d in slow_dmas:
        d.wait()
    fast_dma.wait()


def kernel(frames):
    fast, slow = pl.pallas_call(
        _body,
        in_specs=[pl.BlockSpec(memory_space=pltpu.MemorySpace.HBM)],
        out_specs=[
            pl.BlockSpec(memory_space=pltpu.MemorySpace.HBM),
            pl.BlockSpec(memory_space=pltpu.MemorySpace.HBM),
        ],
        out_shape=[
            jax.ShapeDtypeStruct((_C, _T, _H, _W), frames.dtype),
            jax.ShapeDtypeStruct((_C, _S, _H, _W), frames.dtype),
        ],
        scratch_shapes=[pltpu.SemaphoreType.DMA, pltpu.SemaphoreType.DMA],
    )(frames)
    return slow, fast


# speedup vs baseline: 34.7942x; 34.7942x over previous
import jax
import jax.numpy as jnp
from jax.experimental import pallas as pl
from jax.experimental.pallas import tpu as pltpu

_C, _T, _H, _W = 3, 32, 224, 224
_S = _T // 4
_IDX = tuple((j * (_T - 1)) // (_S - 1) for j in range(_S))


def _gather_body(in_ref, out_ref):
    out_ref[...] = in_ref[...]


def _in_map(j):
    return (0, (j * (_T - 1)) // (_S - 1), 0, 0)


def kernel(frames):
    fast = jnp.copy(frames)
    slow = pl.pallas_call(
        _gather_body,
        grid=(_S,),
        in_specs=[pl.BlockSpec((_C, 1, _H, _W), _in_map)],
        out_specs=pl.BlockSpec((_C, 1, _H, _W), lambda j: (0, j, 0, 0)),
        out_shape=jax.ShapeDtypeStruct((_C, _S, _H, _W), frames.dtype),
    )(frames)
    return slow, fast


# speedup vs baseline: 39.9643x; 1.1486x over previous
import jax
import jax.numpy as jnp
from jax.experimental import pallas as pl
from jax.experimental.pallas import tpu as pltpu

_C, _T, _H, _W = 3, 32, 224, 224
_S = _T // 4
_IDX = tuple((j * (_T - 1)) // (_S - 1) for j in range(_S))


_FPS = 4


def _gather_body(*refs):
    in_refs, out_ref = refs[:_FPS], refs[_FPS]
    for k in range(_FPS):
        out_ref[:, k] = in_refs[k][:, 0]


def _make_in_map(k):
    def in_map(j):
        return (0, ((j * _FPS + k) * (_T - 1)) // (_S - 1), 0, 0)

    return in_map


def kernel(frames):
    fast = jnp.copy(frames)
    slow = pl.pallas_call(
        _gather_body,
        grid=(_S // _FPS,),
        in_specs=[
            pl.BlockSpec((_C, 1, _H, _W), _make_in_map(k)) for k in range(_FPS)
        ],
        out_specs=pl.BlockSpec((_C, _FPS, _H, _W), lambda j: (0, j, 0, 0)),
        out_shape=jax.ShapeDtypeStruct((_C, _S, _H, _W), frames.dtype),
    )(frames, *([frames] * (_FPS - 1)))
    return slow, fast
